# baseline (speedup 1.0000x reference)
import jax
import jax.numpy as jnp
from jax.experimental import pallas as pl


def _body(a_ref, xr_ref, w0_ref, w12_ref, bias_ref, y_ref, *, rows):
    i = pl.program_id(0)
    x1 = jnp.dot(a_ref[...].astype(jnp.bfloat16), xr_ref[...],
                 preferred_element_type=jnp.float32).astype(jnp.bfloat16)
    xblk = xr_ref[pl.ds(i * rows, rows), :]
    n_chunks = x1.shape[1] // 128
    for kc in range(n_chunks):
        ya = jnp.dot(xblk[:, kc * 128:(kc + 1) * 128], w0_ref[...],
                     preferred_element_type=jnp.float32)
        ya = ya + jnp.dot(x1[:, kc * 128:(kc + 1) * 128], w12_ref[...],
                          preferred_element_type=jnp.float32)
        y_ref[:, kc * 256:(kc + 1) * 256] = (
            ya + bias_ref[:, kc * 256:(kc + 1) * 256])


def kernel(x, support, W, b):
    squeeze = x.ndim < 4
    if squeeze:
        x = x[..., None]
    nb, c_in, n, seq = x.shape
    k = nb * seq
    c_out = W.shape[0]

    xr = jnp.transpose(x, (2, 0, 3, 1)).reshape(n, k * c_in).astype(jnp.bfloat16)

    wm = W[:, :, 0, 0]
    w0 = wm[:, :c_in].T
    w12 = (wm[:, c_in:2 * c_in] + wm[:, 2 * c_in:3 * c_in]).T
    eye8 = jnp.eye(8, dtype=jnp.float32)
    w0c = jnp.kron(eye8, w0).astype(jnp.bfloat16)
    w12c = jnp.kron(eye8, w12).astype(jnp.bfloat16)
    bias = jnp.tile(b, k).reshape(1, k * c_out)

    rows = 128
    import functools
    y_flat = pl.pallas_call(
        functools.partial(_body, rows=rows),
        grid=(n // rows,),
        in_specs=[
            pl.BlockSpec((rows, n), lambda i: (i, 0)),
            pl.BlockSpec((n, k * c_in), lambda i: (0, 0)),
            pl.BlockSpec((8 * c_in, 8 * c_out), lambda i: (0, 0)),
            pl.BlockSpec((8 * c_in, 8 * c_out), lambda i: (0, 0)),
            pl.BlockSpec((1, k * c_out), lambda i: (0, 0)),
        ],
        out_specs=pl.BlockSpec((rows, k * c_out), lambda i: (i, 0)),
        out_shape=jax.ShapeDtypeStruct((n, k * c_out), jnp.float32),
    )(support, xr, w0c, w12c, bias)

    y = y_flat.reshape(n, nb, seq, c_out).transpose(1, 3, 0, 2)
    if squeeze:
        y = y[..., 0]
    return y


# speedup vs baseline: 1.1484x; 1.1484x over previous
import jax
import jax.numpy as jnp
from jax.experimental import pallas as pl


def _body(a_ref, xr_ref, w0_ref, w12_ref, bias_ref, y_ref, *, rows):
    i = pl.program_id(0)
    x1 = jnp.dot(a_ref[...].astype(jnp.bfloat16), xr_ref[...],
                 preferred_element_type=jnp.float32).astype(jnp.bfloat16)
    xblk = xr_ref[pl.ds(i * rows, rows), :]
    n_chunks = x1.shape[1] // 128
    for kc in range(n_chunks):
        ya = jnp.dot(xblk[:, kc * 128:(kc + 1) * 128], w0_ref[...],
                     preferred_element_type=jnp.float32)
        ya = ya + jnp.dot(x1[:, kc * 128:(kc + 1) * 128], w12_ref[...],
                          preferred_element_type=jnp.float32)
        y_ref[:, kc * 256:(kc + 1) * 256] = (
            ya + bias_ref[:, kc * 256:(kc + 1) * 256])


def kernel(x, support, W, b):
    squeeze = x.ndim < 4
    if squeeze:
        x = x[..., None]
    nb, c_in, n, seq = x.shape
    k = nb * seq
    c_out = W.shape[0]

    xr = jnp.transpose(x, (2, 0, 3, 1)).reshape(n, k * c_in).astype(jnp.bfloat16)

    wm = W[:, :, 0, 0]
    w0 = wm[:, :c_in].T
    w12 = (wm[:, c_in:2 * c_in] + wm[:, 2 * c_in:3 * c_in]).T
    eye8 = jnp.eye(8, dtype=jnp.float32)
    w0c = jnp.kron(eye8, w0).astype(jnp.bfloat16)
    w12c = jnp.kron(eye8, w12).astype(jnp.bfloat16)
    bias = jnp.tile(b, k).reshape(1, k * c_out)

    rows = 1024
    import functools
    y_flat = pl.pallas_call(
        functools.partial(_body, rows=rows),
        grid=(n // rows,),
        in_specs=[
            pl.BlockSpec((rows, n), lambda i: (i, 0)),
            pl.BlockSpec((n, k * c_in), lambda i: (0, 0)),
            pl.BlockSpec((8 * c_in, 8 * c_out), lambda i: (0, 0)),
            pl.BlockSpec((8 * c_in, 8 * c_out), lambda i: (0, 0)),
            pl.BlockSpec((1, k * c_out), lambda i: (0, 0)),
        ],
        out_specs=pl.BlockSpec((rows, k * c_out), lambda i: (i, 0)),
        out_shape=jax.ShapeDtypeStruct((n, k * c_out), jnp.float32),
    )(support, xr, w0c, w12c, bias)

    y = y_flat.reshape(n, nb, seq, c_out).transpose(1, 3, 0, 2)
    if squeeze:
        y = y[..., 0]
    return y


# speedup vs baseline: 1.1828x; 1.0300x over previous
import jax
import jax.numpy as jnp
from jax.experimental import pallas as pl
from jax.experimental.pallas import tpu as pltpu


def _body(a_ref, xr_ref, w0_ref, w12_ref, bias_ref, y_ref, *, rows):
    i = pl.program_id(0)
    x1 = jnp.dot(a_ref[...].astype(jnp.bfloat16), xr_ref[...],
                 preferred_element_type=jnp.float32).astype(jnp.bfloat16)
    xblk = xr_ref[pl.ds(i * rows, rows), :]
    n_chunks = x1.shape[1] // 128
    for kc in range(n_chunks):
        ya = jnp.dot(xblk[:, kc * 128:(kc + 1) * 128], w0_ref[...],
                     preferred_element_type=jnp.float32)
        ya = ya + jnp.dot(x1[:, kc * 128:(kc + 1) * 128], w12_ref[...],
                          preferred_element_type=jnp.float32)
        y_ref[:, kc * 256:(kc + 1) * 256] = (
            ya + bias_ref[:, kc * 256:(kc + 1) * 256])


def kernel(x, support, W, b):
    squeeze = x.ndim < 4
    if squeeze:
        x = x[..., None]
    nb, c_in, n, seq = x.shape
    k = nb * seq
    c_out = W.shape[0]

    xr = jnp.transpose(x, (2, 0, 3, 1)).reshape(n, k * c_in).astype(jnp.bfloat16)

    wm = W[:, :, 0, 0]
    w0 = wm[:, :c_in].T
    w12 = (wm[:, c_in:2 * c_in] + wm[:, 2 * c_in:3 * c_in]).T
    eye8 = jnp.eye(8, dtype=jnp.float32)
    w0c = jnp.kron(eye8, w0).astype(jnp.bfloat16)
    w12c = jnp.kron(eye8, w12).astype(jnp.bfloat16)
    bias = jnp.tile(b, k).reshape(1, k * c_out)

    rows = 512
    import functools
    y_flat = pl.pallas_call(
        functools.partial(_body, rows=rows),
        grid=(n // rows,),
        in_specs=[
            pl.BlockSpec((rows, n), lambda i: (i, 0)),
            pl.BlockSpec((n, k * c_in), lambda i: (0, 0)),
            pl.BlockSpec((8 * c_in, 8 * c_out), lambda i: (0, 0)),
            pl.BlockSpec((8 * c_in, 8 * c_out), lambda i: (0, 0)),
            pl.BlockSpec((1, k * c_out), lambda i: (0, 0)),
        ],
        out_specs=pl.BlockSpec((rows, k * c_out), lambda i: (i, 0)),
        out_shape=jax.ShapeDtypeStruct((n, k * c_out), jnp.float32),
        compiler_params=pltpu.CompilerParams(
            dimension_semantics=("parallel",),
            vmem_limit_bytes=100 * 1024 * 1024,
        ),
    )(support, xr, w0c, w12c, bias)

    y = y_flat.reshape(n, nb, seq, c_out).transpose(1, 3, 0, 2)
    if squeeze:
        y = y[..., 0]
    return y
